# baseline (speedup 1.0000x reference)
import jax
import jax.numpy as jnp
from jax.experimental import pallas as pl
from jax.experimental.pallas import tpu as pltpu

_CUTOFF = 5.0


def _dot(a, b):
    return jnp.dot(a, b, preferred_element_type=jnp.float32)


def _dotbf(a, b):
    return jnp.dot(a.astype(jnp.bfloat16), b.astype(jnp.bfloat16),
                   preferred_element_type=jnp.float32)


def _fused(T, K, F,
           x_full_ref, x_tile_ref, eqF_ref, eqf_ref, drtab_ref,
           edge_ref, cols_ref, wstack_ref, bstack_ref, wime_ref,
           out_x, out_eqF, out_eqf, out_eqdr):
    N = x_full_ref.shape[1]
    TK = T * K
    W = lambda i: wstack_ref[i]
    bvec = lambda i: bstack_ref[i:i + 1, :]

    def mlp2(x, wi, bi):
        h = _dot(x, W(wi)) + bvec(bi)
        h = h * jax.nn.sigmoid(h)
        return _dot(h, W(wi + 1)) + bvec(bi + 1)

    x_full = x_full_ref[0]
    x_tile = x_tile_ref[0]
    imn_full = mlp2(x_full, 0, 1)
    imn_tile = mlp2(x_tile, 0, 1)

    cols = cols_ref[0]
    pcol = jax.lax.broadcasted_iota(jnp.int32, (6, 5 * F + 2 * N), 1)
    prow = jax.lax.broadcasted_iota(jnp.int32, (6, 5 * F + 2 * N), 0)
    pmat = jnp.where(jnp.minimum(pcol // F, 5) == prow, 1.0, 0.0)
    bcast = _dotbf(cols, pmat)
    d_b = bcast[:, 0:F]
    mask_b = bcast[:, F:2 * F]
    dv_b = bcast[:, 2 * F:5 * F]
    idx_b = bcast[:, 5 * F:5 * F + N]

    edge = edge_ref[0].reshape(TK, edge_ref.shape[3])
    ime = _dotbf(edge, wime_ref[...]) + bvec(0)
    ime = ime * d_b

    iota = jax.lax.broadcasted_iota(jnp.int32, (TK, N), 1).astype(jnp.float32)
    onehot = jnp.where(idx_b == iota, 1.0, 0.0).astype(jnp.bfloat16)
    imn_f = _dotbf(onehot, imn_full)
    dr_f = [_dotbf(onehot, drtab_ref[0, :, c, :]) for c in range(3)]

    imn_i = jnp.broadcast_to(imn_tile[:, None, :], (T, K, F)).reshape(TK, F)
    msg = ime * imn_i * imn_f
    x_new = x_tile + (msg * mask_b).reshape(T, K, F).sum(axis=1)

    s_b = _dot(msg, W(10)) * mask_b
    s1 = s_b[:, 0:1]
    w3 = (s1 * cols[:, 1:2]) * cols[:, 2:5]
    eqF_new = eqF_ref[0] + w3.reshape(T, K, 3).sum(axis=1)

    emf = mlp2(msg, 2, 3)
    h = _dot(msg, W(4))
    h = h * jax.nn.sigmoid(h)
    eme = _dot(h, W(5)) * mask_b

    esc = mlp2(x_new, 6, 5)
    isc = mlp2(x_new, 8, 7)

    dot3 = jnp.zeros((T, F), jnp.float32)
    for c in range(3):
        w_c = s_b * dv_b[:, c * F:(c + 1) * F]
        eqf_c = eqf_ref[0, :, c, :] + (emf * w_c).reshape(T, K, F).sum(axis=1)
        dr_c = drtab_ref[0, pl.ds(pl.program_id(1) * T, T), c, :]
        eqdr_c = dr_c + (eme * dr_f[c]).reshape(T, K, F).sum(axis=1)
        eqdr_c = eqdr_c + esc * eqf_c
        out_eqf[0, :, c, :] = eqf_c
        out_eqdr[0, :, c, :] = eqdr_c
        dot3 = dot3 + eqdr_c * eqf_c

    out_x[0] = x_new + isc * dot3
    out_eqF[0] = eqF_new


def kernel(invariant_node, equivariant_node_F, equivariant_node_f,
           equivariant_node_dr, invariant_edge, neighbor_mask, distances,
           distance_vectors, neighbor_indices, params):
    B, N, F = invariant_node.shape
    K = neighbor_indices.shape[-1]
    NB = invariant_edge.shape[-1]
    T = 64
    TK = T * K
    p = params

    cols = jnp.concatenate([
        (0.5 * (jnp.cos(jnp.pi * distances / _CUTOFF) + 1.0)
         * (distances < _CUTOFF)).reshape(B, N * K, 1),
        neighbor_mask.reshape(B, N * K, 1),
        distance_vectors.reshape(B, N * K, 3),
        neighbor_indices.astype(jnp.float32).reshape(B, N * K, 1),
    ], axis=-1)

    wstack = jnp.stack([
        p["imn_W1"].T, p["imn_W2"].T, p["emf_W1"].T, p["emf_W2"].T,
        p["eme_W1"].T, p["eme_W2"].T, p["esc_W1"].T, p["esc_W2"].T,
        p["isc_W1"].T, p["isc_W2"].T,
        jnp.broadcast_to(p["W_emc"].T, (F, F)),
    ])
    bstack = jnp.stack([
        p["b_ime"], p["imn_b1"], p["imn_b2"], p["emf_b1"], p["emf_b2"],
        p["esc_b1"], p["esc_b2"], p["isc_b1"], p["isc_b2"],
    ])

    def wspec(w):
        shp = w.shape
        return pl.BlockSpec(shp, lambda b, t: (0,) * len(shp))

    grid = (B, N // T)
    in_specs = [
        pl.BlockSpec((1, N, F), lambda b, t: (b, 0, 0)),
        pl.BlockSpec((1, T, F), lambda b, t: (b, t, 0)),
        pl.BlockSpec((1, T, 3), lambda b, t: (b, t, 0)),
        pl.BlockSpec((1, T, 3, F), lambda b, t: (b, t, 0, 0)),
        pl.BlockSpec((1, N, 3, F), lambda b, t: (b, 0, 0, 0)),
        pl.BlockSpec((1, T, K, NB), lambda b, t: (b, t, 0, 0)),
        pl.BlockSpec((1, TK, 6), lambda b, t: (b, t, 0)),
        wspec(wstack), wspec(bstack), wspec(p["W_ime"].T),
    ]

    out_specs = [
        pl.BlockSpec((1, T, F), lambda b, t: (b, t, 0)),
        pl.BlockSpec((1, T, 3), lambda b, t: (b, t, 0)),
        pl.BlockSpec((1, T, 3, F), lambda b, t: (b, t, 0, 0)),
        pl.BlockSpec((1, T, 3, F), lambda b, t: (b, t, 0, 0)),
    ]
    out_shapes = [
        jax.ShapeDtypeStruct((B, N, F), jnp.float32),
        jax.ShapeDtypeStruct((B, N, 3), jnp.float32),
        jax.ShapeDtypeStruct((B, N, 3, F), jnp.float32),
        jax.ShapeDtypeStruct((B, N, 3, F), jnp.float32),
    ]

    import functools
    fn = functools.partial(_fused, T, K, F)
    outs = pl.pallas_call(
        fn,
        grid=grid,
        in_specs=in_specs,
        out_specs=out_specs,
        out_shape=out_shapes,
        compiler_params=pltpu.CompilerParams(
            dimension_semantics=("parallel", "parallel")),
    )(invariant_node, invariant_node, equivariant_node_F, equivariant_node_f,
      equivariant_node_dr, invariant_edge, cols,
      wstack, bstack, p["W_ime"].T)

    return outs[0], outs[1], outs[2], outs[3]


# speedup vs baseline: 1.4250x; 1.4250x over previous
import functools

import jax
import jax.numpy as jnp
from jax.experimental import pallas as pl
from jax.experimental.pallas import tpu as pltpu

_CUTOFF = 5.0


def _mm(a, b):
    return jnp.dot(a, b, preferred_element_type=jnp.float32)


def _fwd(T, K, F, N, NB,
         x_full_ref, x_tile_ref, eqf_ref, drfull_ref, drtile_ref,
         edge_ref, cut_ref, mask_ref, dvec_ref, idx_ref,
         w_ref, b_ref, wime_ref,
         out_x, out_eqFt, out_eqf, out_eqdr, tab_ref):
    bf = jnp.bfloat16

    wb = w_ref[...]

    def bias(i, width):
        return b_ref[i][:, :width]

    def mlp2T(xT, wi, bi):
        width = xT.shape[1]
        h = _mm(wb[wi], xT) + bias(bi, width)
        h = h * jax.nn.sigmoid(h)
        return _mm(wb[wi + 1], h) + bias(bi + 1, width)

    x_tile = x_tile_ref[0]
    x_tileT = jnp.transpose(x_tile)
    x_fullT = jnp.transpose(x_full_ref[0])
    tab_ref[0] = mlp2T(x_fullT, 0, 1)
    imn_iT = mlp2T(x_tileT, 0, 1)
    tab_ref[1] = jnp.transpose(drfull_ref[0, 0])
    tab_ref[2] = jnp.transpose(drfull_ref[0, 1])
    tab_ref[3] = jnp.transpose(drfull_ref[0, 2])
    imnT_bf = tab_ref[0]
    drT_bf = [tab_ref[1], tab_ref[2], tab_ref[3]]
    wime = wime_ref[...]
    zero = jnp.zeros((F, T), jnp.float32)

    def body(k, carry):
        ax, aF, af0, af1, af2, ad0, ad1, ad2 = carry
        ek = edge_ref[0, :, pl.ds(k, 1), :].reshape(NB, T)
        cutk = cut_ref[0, pl.ds(k, 1), :]
        wm = mask_ref[0, pl.ds(k, 1), :]
        dvk = dvec_ref[0, :, pl.ds(k, 1), :].reshape(3, T)
        idxk = idx_ref[0, pl.ds(k, 1), :]

        ime = (_mm(wime, ek) + bias(0, T)) * cutk
        iota = jax.lax.broadcasted_iota(
            jnp.int32, (N, T), 0).astype(jnp.float32)
        oh = jnp.where(iota == idxk, 1.0, 0.0)
        imn_f = _mm(imnT_bf, oh)
        msg = ime * imn_iT * imn_f
        ax = ax + msg * wm
        s_bT = _mm(wb[10], msg)
        swm = s_bT[0:1, :] * wm
        aF = aF + swm * dvk
        h = _mm(wb[2], msg) + bias(3, T)
        h = h * jax.nn.sigmoid(h)
        emf = _mm(wb[3], h) + bias(4, T)
        swb = s_bT * wm
        af0 = af0 + emf * (swb * dvk[0:1, :])
        af1 = af1 + emf * (swb * dvk[1:2, :])
        af2 = af2 + emf * (swb * dvk[2:3, :])
        h2 = _mm(wb[4], msg)
        h2 = h2 * jax.nn.sigmoid(h2)
        eme = _mm(wb[5], h2) * wm
        ad0 = ad0 + eme * _mm(drT_bf[0], oh)
        ad1 = ad1 + eme * _mm(drT_bf[1], oh)
        ad2 = ad2 + eme * _mm(drT_bf[2], oh)
        return ax, aF, af0, af1, af2, ad0, ad1, ad2

    carry = (zero, jnp.zeros((3, T), jnp.float32),
             zero, zero, zero, zero, zero, zero)
    for k in range(K):
        carry = body(k, carry)
    ax, aF, af0, af1, af2, ad0, ad1, ad2 = carry

    x_newT = x_tileT + ax
    escT = mlp2T(x_newT, 6, 5)
    iscT = mlp2T(x_newT, 8, 7)

    x_new = x_tile + jnp.transpose(ax)
    isc = jnp.transpose(iscT)
    out_eqFt[0] = aF

    afs = (af0, af1, af2)
    ads = (ad0, ad1, ad2)
    dot3 = jnp.zeros((T, F), jnp.float32)
    for c in range(3):
        eqf_c = jnp.transpose(afs[c]) + eqf_ref[0, c]
        eqdr_c = drtile_ref[0, c] + jnp.transpose(
            ads[c] + escT * afs[c])
        out_eqf[0, :, c, :] = eqf_c
        out_eqdr[0, :, c, :] = eqdr_c
        dot3 = dot3 + eqdr_c * eqf_c

    out_x[0] = x_new + isc * dot3


def kernel(invariant_node, equivariant_node_F, equivariant_node_f,
           equivariant_node_dr, invariant_edge, neighbor_mask, distances,
           distance_vectors, neighbor_indices, params):
    B, N, F = invariant_node.shape
    K = neighbor_indices.shape[-1]
    NB = invariant_edge.shape[-1]
    T = 256
    p = params

    edge_t = jnp.transpose(invariant_edge, (0, 3, 2, 1))
    cut = (0.5 * (jnp.cos(jnp.pi * distances / _CUTOFF) + 1.0)
           * (distances < _CUTOFF))
    cut_t = jnp.transpose(cut, (0, 2, 1))
    mask_t = jnp.transpose(neighbor_mask, (0, 2, 1))
    dvec_t = jnp.transpose(distance_vectors, (0, 3, 2, 1))
    idx_t = jnp.transpose(neighbor_indices.astype(jnp.float32), (0, 2, 1))
    eqf_t = jnp.transpose(equivariant_node_f, (0, 2, 1, 3))
    eqdr_t = jnp.transpose(equivariant_node_dr, (0, 2, 1, 3))

    wstack = jnp.stack([
        p["imn_W1"], p["imn_W2"], p["emf_W1"], p["emf_W2"],
        p["eme_W1"], p["eme_W2"], p["esc_W1"], p["esc_W2"],
        p["isc_W1"], p["isc_W2"],
        jnp.broadcast_to(p["W_emc"], (F, F)),
    ])
    bstack = jnp.broadcast_to(jnp.stack([
        p["b_ime"], p["imn_b1"], p["imn_b2"], p["emf_b1"], p["emf_b2"],
        p["esc_b1"], p["esc_b2"], p["isc_b1"], p["isc_b2"],
    ])[:, :, None], (9, F, N))
    wime = p["W_ime"]

    def cspec(shp):
        return pl.BlockSpec(shp, lambda b, t: (0,) * len(shp))

    grid = (B, N // T)
    in_specs = [
        pl.BlockSpec((1, N, F), lambda b, t: (b, 0, 0)),
        pl.BlockSpec((1, T, F), lambda b, t: (b, t, 0)),
        pl.BlockSpec((1, 3, T, F), lambda b, t: (b, 0, t, 0)),
        pl.BlockSpec((1, 3, N, F), lambda b, t: (b, 0, 0, 0)),
        pl.BlockSpec((1, 3, T, F), lambda b, t: (b, 0, t, 0)),
        pl.BlockSpec((1, NB, K, T), lambda b, t: (b, 0, 0, t)),
        pl.BlockSpec((1, K, T), lambda b, t: (b, 0, t)),
        pl.BlockSpec((1, K, T), lambda b, t: (b, 0, t)),
        pl.BlockSpec((1, 3, K, T), lambda b, t: (b, 0, 0, t)),
        pl.BlockSpec((1, K, T), lambda b, t: (b, 0, t)),
        cspec((11, F, F)), cspec((9, F, N)), cspec((F, NB)),
    ]
    out_specs = [
        pl.BlockSpec((1, T, F), lambda b, t: (b, t, 0)),
        pl.BlockSpec((1, 3, T), lambda b, t: (b, 0, t)),
        pl.BlockSpec((1, T, 3, F), lambda b, t: (b, t, 0, 0)),
        pl.BlockSpec((1, T, 3, F), lambda b, t: (b, t, 0, 0)),
    ]
    out_shapes = [
        jax.ShapeDtypeStruct((B, N, F), jnp.float32),
        jax.ShapeDtypeStruct((B, 3, N), jnp.float32),
        jax.ShapeDtypeStruct((B, N, 3, F), jnp.float32),
        jax.ShapeDtypeStruct((B, N, 3, F), jnp.float32),
    ]

    fn = functools.partial(_fwd, T, K, F, N, NB)
    outs = pl.pallas_call(
        fn,
        grid=grid,
        in_specs=in_specs,
        out_specs=out_specs,
        out_shape=out_shapes,
        compiler_params=pltpu.CompilerParams(
            dimension_semantics=("parallel", "parallel")),
        scratch_shapes=[pltpu.VMEM((4, F, N), jnp.float32)],
    )(invariant_node, invariant_node, eqf_t, eqdr_t, eqdr_t,
      edge_t, cut_t, mask_t, dvec_t, idx_t, wstack, bstack, wime)

    return (outs[0],
            jnp.transpose(outs[1], (0, 2, 1)) + equivariant_node_F,
            outs[2], outs[3])
